# baseline (speedup 1.0000x reference)
import functools

import jax
import jax.numpy as jnp
import numpy as np
from jax import lax
from jax.experimental import pallas as pl
from jax.experimental.pallas import tpu as pltpu
from jax.experimental.pallas import tpu_sc as plsc

B = 2
L = 2048
D_MODEL = 1024
N_HEADS = 16
D_K = D_MODEL // N_HEADS
TOP_K = 8
BH = B * N_HEADS
NF = L // 2 + 1
FPAD = 1152

_f = np.arange(FPAD, dtype=np.float64)
_t = np.arange(L, dtype=np.float64)
_theta = (2.0 * np.pi / L) * np.outer(_f, _t)
_mask = (_f < NF)[:, None]
_FWC = np.where(_mask, np.cos(_theta), 0.0).astype(np.float32)
_FWS = np.where(_mask, np.sin(_theta), 0.0).astype(np.float32)
_w = np.where((_f >= 1) & (_f <= NF - 2), 2.0, 1.0) * (_f < NF)
_IDC = (_w[:, None] * np.cos(_theta)).astype(np.float32)
_IDS = (_w[:, None] * np.sin(_theta)).astype(np.float32)
_ED = np.zeros((D_MODEL, N_HEADS), dtype=np.float32)
for _c in range(D_MODEL):
    _ED[_c, _c // D_K] = 1.0 / (L * D_K)

_HI = lax.Precision.HIGHEST


def _mm_kernel(x_ref, y_ref, o_ref, *, precision):
    @pl.when(pl.program_id(2) == 0)
    def _():
        o_ref[...] = jnp.zeros_like(o_ref)

    o_ref[...] += jnp.dot(x_ref[...], y_ref[...],
                          preferred_element_type=jnp.float32,
                          precision=precision)


def _mm_bias_kernel(x_ref, y_ref, b_ref, o_ref, *, precision):
    @pl.when(pl.program_id(2) == 0)
    def _():
        o_ref[...] = jnp.broadcast_to(b_ref[...], o_ref.shape)

    o_ref[...] += jnp.dot(x_ref[...], y_ref[...],
                          preferred_element_type=jnp.float32,
                          precision=precision)


def _matmul(x, y, bias=None, bm=512, bn=512, bk=512, precision=_HI):
    M, K = x.shape
    _, N = y.shape
    bm, bn, bk = min(bm, M), min(bn, N), min(bk, K)
    grid = (M // bm, N // bn, K // bk)
    in_specs = [
        pl.BlockSpec((bm, bk), lambda i, j, k: (i, k)),
        pl.BlockSpec((bk, bn), lambda i, j, k: (k, j)),
    ]
    args = [x, y]
    if bias is None:
        body = functools.partial(_mm_kernel, precision=precision)
    else:
        body = functools.partial(_mm_bias_kernel, precision=precision)
        in_specs.append(pl.BlockSpec((1, bn), lambda i, j, k: (0, j)))
        args.append(bias.reshape(1, N))
    return pl.pallas_call(
        body,
        grid=grid,
        in_specs=in_specs,
        out_specs=pl.BlockSpec((bm, bn), lambda i, j, k: (i, j)),
        out_shape=jax.ShapeDtypeStruct((M, N), jnp.float32),
        compiler_params=pltpu.CompilerParams(
            dimension_semantics=("parallel", "parallel", "arbitrary")),
    )(*args)


def _fwd_kernel(fc_ref, fs_ref, q_ref, k_ref,
                qc_ref, qs_ref, kc_ref, ks_ref):
    @pl.when(pl.program_id(3) == 0)
    def _():
        qc_ref[...] = jnp.zeros_like(qc_ref)
        qs_ref[...] = jnp.zeros_like(qs_ref)
        kc_ref[...] = jnp.zeros_like(kc_ref)
        ks_ref[...] = jnp.zeros_like(ks_ref)

    fcb, fsb = fc_ref[...], fs_ref[...]
    qb, kb = q_ref[0], k_ref[0]
    dot = functools.partial(jnp.dot, preferred_element_type=jnp.float32,
                            precision=_HI)
    qc_ref[0] += dot(fcb, qb)
    qs_ref[0] += dot(fsb, qb)
    kc_ref[0] += dot(fcb, kb)
    ks_ref[0] += dot(fsb, kb)


def _fwd_transforms(fwc, fws, q3, k3, bm=384, bn=512, bk=512):
    grid = (B, FPAD // bm, D_MODEL // bn, L // bk)
    fspec = pl.BlockSpec((bm, bk), lambda b, i, j, k: (i, k))
    xspec = pl.BlockSpec((1, bk, bn), lambda b, i, j, k: (b, k, j))
    ospec = pl.BlockSpec((1, bm, bn), lambda b, i, j, k: (b, i, j))
    oshape = jax.ShapeDtypeStruct((B, FPAD, D_MODEL), jnp.float32)
    return pl.pallas_call(
        _fwd_kernel,
        grid=grid,
        in_specs=[fspec, fspec, xspec, xspec],
        out_specs=[ospec] * 4,
        out_shape=[oshape] * 4,
        compiler_params=pltpu.CompilerParams(
            dimension_semantics=("parallel", "parallel", "parallel",
                                 "arbitrary")),
    )(fwc, fws, q3, k3)


def _spectrum_kernel(qc_ref, qs_ref, kc_ref, ks_ref, ed_ref, sr_ref, si_ref):
    qc, qs = qc_ref[0], qs_ref[0]
    kc, ks = kc_ref[0], ks_ref[0]
    ed = ed_ref[...]
    dot = functools.partial(jnp.dot, preferred_element_type=jnp.float32,
                            precision=_HI)
    sr_ref[0] = dot(qc * kc + qs * ks, ed)
    si_ref[0] = dot(qc * ks - qs * kc, ed)


def _cross_spectrum(qc, qs, kc, ks, ed, bm=384):
    grid = (B, FPAD // bm)
    spec = pl.BlockSpec((1, bm, D_MODEL), lambda b, i: (b, i, 0))
    return pl.pallas_call(
        _spectrum_kernel,
        grid=grid,
        in_specs=[spec, spec, spec, spec,
                  pl.BlockSpec((D_MODEL, N_HEADS), lambda b, i: (0, 0))],
        out_specs=[pl.BlockSpec((1, bm, N_HEADS), lambda b, i: (b, i, 0))] * 2,
        out_shape=[jax.ShapeDtypeStruct((B, FPAD, N_HEADS), jnp.float32)] * 2,
        compiler_params=pltpu.CompilerParams(
            dimension_semantics=("parallel", "parallel")),
    )(qc, qs, kc, ks, ed)


def _idft_kernel(srt_ref, sit_ref, idc_ref, ids_ref, o_ref):
    @pl.when(pl.program_id(2) == 0)
    def _():
        o_ref[...] = jnp.zeros_like(o_ref)

    dot = functools.partial(jnp.dot, preferred_element_type=jnp.float32,
                            precision=_HI)
    o_ref[0] += (dot(srt_ref[0], idc_ref[...])
                 - dot(sit_ref[0], ids_ref[...]))


def _idft(srt, sit, idc, ids, bn=512, bk=384):
    grid = (B, L // bn, FPAD // bk)
    sspec = pl.BlockSpec((1, N_HEADS, bk), lambda b, j, k: (b, 0, k))
    fspec = pl.BlockSpec((bk, bn), lambda b, j, k: (k, j))
    return pl.pallas_call(
        _idft_kernel,
        grid=grid,
        in_specs=[sspec, sspec, fspec, fspec],
        out_specs=pl.BlockSpec((1, N_HEADS, bn), lambda b, j, k: (b, 0, j)),
        out_shape=jax.ShapeDtypeStruct((B, N_HEADS, L), jnp.float32),
        compiler_params=pltpu.CompilerParams(
            dimension_semantics=("parallel", "parallel", "arbitrary")),
    )(srt, sit, idc, ids)


def _agg_kernel(corr_ref, v_ref, o_ref, scratch):
    vb = v_ref[0]
    scratch[0:L, :] = vb
    scratch[L:2 * L, :] = vb
    cpair = corr_ref[...].reshape(2, L)
    iota = lax.broadcasted_iota(jnp.int32, (1, L), 1)
    for i in range(2):
        cv = cpair[i:i + 1, :]
        vals, idxs = [], []
        for _ in range(TOP_K):
            m = jnp.max(cv)
            idx = jnp.min(jnp.where(cv == m, iota, L))
            vals.append(m)
            idxs.append(idx)
            cv = jnp.where(iota == idx, -jnp.inf, cv)
        exps = [jnp.exp(val - vals[0]) for val in vals]
        total = exps[0]
        for e in exps[1:]:
            total = total + e
        sl = slice(i * D_K, (i + 1) * D_K)
        acc = (exps[0] / total) * scratch[pl.ds(L - idxs[0], L), sl]
        for j in range(1, TOP_K):
            acc += (exps[j] / total) * scratch[pl.ds(L - idxs[j], L), sl]
        o_ref[0, :, sl] = acc


def _topk_agg(corr4, v3):
    return pl.pallas_call(
        _agg_kernel,
        grid=(B, N_HEADS // 2),
        in_specs=[
            pl.BlockSpec((1, 1, 2, L), lambda b, hp: (b, hp, 0, 0)),
            pl.BlockSpec((1, L, 2 * D_K), lambda b, hp: (b, 0, hp)),
        ],
        out_specs=pl.BlockSpec((1, L, 2 * D_K), lambda b, hp: (b, 0, hp)),
        out_shape=jax.ShapeDtypeStruct((B, L, D_MODEL), jnp.float32),
        scratch_shapes=[pltpu.VMEM((2 * L, 2 * D_K), jnp.float32)],
        compiler_params=pltpu.CompilerParams(
            dimension_semantics=("parallel", "parallel")),
    )(corr4, v3)


_SC_T = 128
_SC_NCH = L // _SC_T
_SC_LANES = 16


def _sc_agg_body(corr_hbm, v2_hbm, out_hbm,
                 corr_v, buf_v, acc_v, sem):
    wid = lax.axis_index("s") * 2 + lax.axis_index("c")

    pltpu.sync_copy(corr_hbm.at[wid], corr_v)

    def topk_body(i, carry):
        ck, ci = carry
        vals = corr_v[0, pl.ds(i * _SC_LANES, _SC_LANES)]
        idxs = lax.iota(jnp.int32, _SC_LANES) + i * _SC_LANES
        sv, si = plsc.sort_key_val(vals, idxs, descending=True)
        rv = lax.rev(sv, (0,))
        ri = lax.rev(si, (0,))
        take_old = ck >= rv
        mk = jnp.maximum(ck, rv)
        mi = jnp.where(take_old, ci, ri)
        return tuple(plsc.sort_key_val(mk, mi, descending=True))

    ck0 = jnp.full((_SC_LANES,), -jnp.inf, jnp.float32)
    ci0 = jnp.zeros((_SC_LANES,), jnp.int32)
    ck, ci = lax.fori_loop(0, L // _SC_LANES, topk_body, (ck0, ci0))

    m = lax.reduce_max(ck, (0,))
    lanes = lax.iota(jnp.int32, _SC_LANES)
    e = jnp.where(lanes < TOP_K, jnp.exp(ck - m), 0.0)
    s = lax.reduce_sum(e, (0,))
    w = e / s
    ws = [w[j] for j in range(TOP_K)]
    ds = [ci[j] for j in range(TOP_K)]
    bases = [L - d for d in ds]
    rs = [b % 8 for b in bases]
    starts = [b - r for b, r in zip(bases, rs)]

    for c in range(_SC_NCH):
        copies = []
        for j in range(TOP_K):
            st = pl.multiple_of(starts[j] + c * _SC_T, 8)
            src = v2_hbm.at[wid, pl.ds(st, _SC_T + 8), :]
            copies.append(pltpu.async_copy(src, buf_v.at[j], sem))
        for cp in copies:
            cp.wait()

        def chunk_body(t, _):
            for l in range(D_K // _SC_LANES):
                sl = pl.ds(l * _SC_LANES, _SC_LANES)
                a = ws[0] * buf_v[0, t + rs[0], sl]
                for j in range(1, TOP_K):
                    a += ws[j] * buf_v[j, t + rs[j], sl]
                acc_v[t, sl] = a
            return 0

        lax.fori_loop(0, _SC_T, chunk_body, 0)
        pltpu.sync_copy(acc_v,
                        out_hbm.at[wid, pl.ds(c * _SC_T, _SC_T), :])


def _topk_agg_sc(corr2, v2d):
    mesh = plsc.VectorSubcoreMesh(core_axis_name="c", subcore_axis_name="s")
    f = functools.partial(
        pl.kernel,
        out_type=jax.ShapeDtypeStruct((BH, L, D_K), jnp.float32),
        mesh=mesh,
        compiler_params=pltpu.CompilerParams(needs_layout_passes=False,
                                             use_tc_tiling_on_sc=False),
        scratch_types=[
            pltpu.VMEM((1, L), jnp.float32),
            pltpu.VMEM((TOP_K, _SC_T + 8, D_K), jnp.float32),
            pltpu.VMEM((_SC_T, D_K), jnp.float32),
            pltpu.SemaphoreType.DMA,
        ],
    )(_sc_agg_body)
    return f(corr2, v2d)


def kernel(queries, keys, values, Wq, bq, Wk, bk, Wv, bv, Wo, bo):
    fwc = jnp.asarray(_FWC)
    fws = jnp.asarray(_FWS)
    idc = jnp.asarray(_IDC)
    ids = jnp.asarray(_IDS)
    ed = jnp.asarray(_ED)

    q = _matmul(queries.reshape(B * L, D_MODEL), Wq.T, bq, bk=1024,
                precision=lax.Precision.DEFAULT)
    k = _matmul(keys.reshape(B * L, D_MODEL), Wk.T, bk, bk=1024,
                precision=lax.Precision.DEFAULT)
    v = _matmul(values.reshape(B * L, D_MODEL), Wv.T, bv, bk=1024,
                precision=lax.Precision.DEFAULT)

    q3 = q.reshape(B, L, D_MODEL)
    k3 = k.reshape(B, L, D_MODEL)
    v3 = v.reshape(B, L, D_MODEL)

    qc, qs, kc, ks = _fwd_transforms(fwc, fws, q3, k3)
    sr, si = _cross_spectrum(qc, qs, kc, ks, ed)
    srt = sr.transpose(0, 2, 1)
    sit = si.transpose(0, 2, 1)
    corr = _idft(srt, sit, idc, ids)

    corr2 = corr.reshape(BH, 1, L)
    vh = v3.reshape(B, L, N_HEADS, D_K).transpose(0, 2, 1, 3)
    vh = vh.reshape(BH, L, D_K)
    v2d = jnp.concatenate([vh, vh, vh[:, :8]], axis=1)
    oh = _topk_agg_sc(corr2, v2d)
    outf = oh.reshape(B, N_HEADS, L, D_K).transpose(0, 2, 1, 3)
    out = _matmul(outf.reshape(B * L, D_MODEL), Wo.T, bo, bk=1024,
                  precision=lax.Precision.DEFAULT)
    return out.reshape(B, L, D_MODEL)


# speedup vs baseline: 1.6545x; 1.6545x over previous
import functools

import jax
import jax.numpy as jnp
import numpy as np
from jax import lax
from jax.experimental import pallas as pl
from jax.experimental.pallas import tpu as pltpu
from jax.experimental.pallas import tpu_sc as plsc

B = 2
L = 2048
D_MODEL = 1024
N_HEADS = 16
D_K = D_MODEL // N_HEADS
TOP_K = 8
BH = B * N_HEADS
NF = L // 2 + 1
FPAD = 1152

_f = np.arange(FPAD, dtype=np.float64)
_t = np.arange(L, dtype=np.float64)
_theta = (2.0 * np.pi / L) * np.outer(_f, _t)
_mask = (_f < NF)[:, None]
_FWC = np.where(_mask, np.cos(_theta), 0.0).astype(np.float32)
_FWS = np.where(_mask, np.sin(_theta), 0.0).astype(np.float32)
_w = np.where((_f >= 1) & (_f <= NF - 2), 2.0, 1.0) * (_f < NF)
_IDC = (_w[:, None] * np.cos(_theta)).astype(np.float32)
_IDS = (_w[:, None] * np.sin(_theta)).astype(np.float32)
_ED = np.zeros((D_MODEL, N_HEADS), dtype=np.float32)
for _c in range(D_MODEL):
    _ED[_c, _c // D_K] = 1.0 / (L * D_K)

_HI = lax.Precision.HIGHEST


def _mm_kernel(x_ref, y_ref, o_ref, *, precision):
    @pl.when(pl.program_id(2) == 0)
    def _():
        o_ref[...] = jnp.zeros_like(o_ref)

    o_ref[...] += jnp.dot(x_ref[...], y_ref[...],
                          preferred_element_type=jnp.float32,
                          precision=precision)


def _mm_bias_kernel(x_ref, y_ref, b_ref, o_ref, *, precision):
    @pl.when(pl.program_id(2) == 0)
    def _():
        o_ref[...] = jnp.broadcast_to(b_ref[...], o_ref.shape)

    o_ref[...] += jnp.dot(x_ref[...], y_ref[...],
                          preferred_element_type=jnp.float32,
                          precision=precision)


def _matmul(x, y, bias=None, bm=512, bn=512, bk=512, precision=_HI):
    M, K = x.shape
    _, N = y.shape
    bm, bn, bk = min(bm, M), min(bn, N), min(bk, K)
    grid = (M // bm, N // bn, K // bk)
    in_specs = [
        pl.BlockSpec((bm, bk), lambda i, j, k: (i, k)),
        pl.BlockSpec((bk, bn), lambda i, j, k: (k, j)),
    ]
    args = [x, y]
    if bias is None:
        body = functools.partial(_mm_kernel, precision=precision)
    else:
        body = functools.partial(_mm_bias_kernel, precision=precision)
        in_specs.append(pl.BlockSpec((1, bn), lambda i, j, k: (0, j)))
        args.append(bias.reshape(1, N))
    return pl.pallas_call(
        body,
        grid=grid,
        in_specs=in_specs,
        out_specs=pl.BlockSpec((bm, bn), lambda i, j, k: (i, j)),
        out_shape=jax.ShapeDtypeStruct((M, N), jnp.float32),
        compiler_params=pltpu.CompilerParams(
            dimension_semantics=("parallel", "parallel", "arbitrary")),
    )(*args)


def _fwd_kernel(fc_ref, fs_ref, q_ref, k_ref,
                qc_ref, qs_ref, kc_ref, ks_ref):
    @pl.when(pl.program_id(3) == 0)
    def _():
        qc_ref[...] = jnp.zeros_like(qc_ref)
        qs_ref[...] = jnp.zeros_like(qs_ref)
        kc_ref[...] = jnp.zeros_like(kc_ref)
        ks_ref[...] = jnp.zeros_like(ks_ref)

    fcb, fsb = fc_ref[...], fs_ref[...]
    qb, kb = q_ref[0], k_ref[0]
    dot = functools.partial(jnp.dot, preferred_element_type=jnp.float32,
                            precision=_HI)
    qc_ref[0] += dot(fcb, qb)
    qs_ref[0] += dot(fsb, qb)
    kc_ref[0] += dot(fcb, kb)
    ks_ref[0] += dot(fsb, kb)


def _fwd_transforms(fwc, fws, q3, k3, bm=384, bn=512, bk=512):
    grid = (B, FPAD // bm, D_MODEL // bn, L // bk)
    fspec = pl.BlockSpec((bm, bk), lambda b, i, j, k: (i, k))
    xspec = pl.BlockSpec((1, bk, bn), lambda b, i, j, k: (b, k, j))
    ospec = pl.BlockSpec((1, bm, bn), lambda b, i, j, k: (b, i, j))
    oshape = jax.ShapeDtypeStruct((B, FPAD, D_MODEL), jnp.float32)
    return pl.pallas_call(
        _fwd_kernel,
        grid=grid,
        in_specs=[fspec, fspec, xspec, xspec],
        out_specs=[ospec] * 4,
        out_shape=[oshape] * 4,
        compiler_params=pltpu.CompilerParams(
            dimension_semantics=("parallel", "parallel", "parallel",
                                 "arbitrary")),
    )(fwc, fws, q3, k3)


def _spectrum_kernel(qc_ref, qs_ref, kc_ref, ks_ref, ed_ref, sr_ref, si_ref):
    qc, qs = qc_ref[0], qs_ref[0]
    kc, ks = kc_ref[0], ks_ref[0]
    ed = ed_ref[...]
    dot = functools.partial(jnp.dot, preferred_element_type=jnp.float32,
                            precision=_HI)
    sr_ref[0] = dot(qc * kc + qs * ks, ed)
    si_ref[0] = dot(qc * ks - qs * kc, ed)


def _cross_spectrum(qc, qs, kc, ks, ed, bm=384):
    grid = (B, FPAD // bm)
    spec = pl.BlockSpec((1, bm, D_MODEL), lambda b, i: (b, i, 0))
    return pl.pallas_call(
        _spectrum_kernel,
        grid=grid,
        in_specs=[spec, spec, spec, spec,
                  pl.BlockSpec((D_MODEL, N_HEADS), lambda b, i: (0, 0))],
        out_specs=[pl.BlockSpec((1, bm, N_HEADS), lambda b, i: (b, i, 0))] * 2,
        out_shape=[jax.ShapeDtypeStruct((B, FPAD, N_HEADS), jnp.float32)] * 2,
        compiler_params=pltpu.CompilerParams(
            dimension_semantics=("parallel", "parallel")),
    )(qc, qs, kc, ks, ed)


def _idft_kernel(srt_ref, sit_ref, idc_ref, ids_ref, o_ref):
    @pl.when(pl.program_id(2) == 0)
    def _():
        o_ref[...] = jnp.zeros_like(o_ref)

    dot = functools.partial(jnp.dot, preferred_element_type=jnp.float32,
                            precision=_HI)
    o_ref[0] += (dot(srt_ref[0], idc_ref[...])
                 - dot(sit_ref[0], ids_ref[...]))


def _idft(srt, sit, idc, ids, bn=512, bk=384):
    grid = (B, L // bn, FPAD // bk)
    sspec = pl.BlockSpec((1, N_HEADS, bk), lambda b, j, k: (b, 0, k))
    fspec = pl.BlockSpec((bk, bn), lambda b, j, k: (k, j))
    return pl.pallas_call(
        _idft_kernel,
        grid=grid,
        in_specs=[sspec, sspec, fspec, fspec],
        out_specs=pl.BlockSpec((1, N_HEADS, bn), lambda b, j, k: (b, 0, j)),
        out_shape=jax.ShapeDtypeStruct((B, N_HEADS, L), jnp.float32),
        compiler_params=pltpu.CompilerParams(
            dimension_semantics=("parallel", "parallel", "arbitrary")),
    )(srt, sit, idc, ids)


def _agg_kernel(w_ref, d_ref, v_ref, o_ref, scratch):
    vb = v_ref[0]
    scratch[0:L, :] = vb
    scratch[L:2 * L, :] = vb
    wpair = w_ref[...].reshape(2, _SC_LANES)
    dpair = d_ref[...].reshape(2, _SC_LANES)
    for i in range(2):
        sl = slice(i * D_K, (i + 1) * D_K)
        acc = wpair[i, 0] * scratch[pl.ds(L - dpair[i, 0], L), sl]
        for j in range(1, TOP_K):
            acc += wpair[i, j] * scratch[pl.ds(L - dpair[i, j], L), sl]
        o_ref[0, :, sl] = acc


def _topk_agg(w4, d4, v3):
    return pl.pallas_call(
        _agg_kernel,
        grid=(B, N_HEADS // 2),
        in_specs=[
            pl.BlockSpec((1, 1, 2, _SC_LANES), lambda b, hp: (b, hp, 0, 0)),
            pl.BlockSpec((1, 1, 2, _SC_LANES), lambda b, hp: (b, hp, 0, 0)),
            pl.BlockSpec((1, L, 2 * D_K), lambda b, hp: (b, 0, hp)),
        ],
        out_specs=pl.BlockSpec((1, L, 2 * D_K), lambda b, hp: (b, 0, hp)),
        out_shape=jax.ShapeDtypeStruct((B, L, D_MODEL), jnp.float32),
        scratch_shapes=[pltpu.VMEM((2 * L, 2 * D_K), jnp.float32)],
        compiler_params=pltpu.CompilerParams(
            dimension_semantics=("parallel", "parallel")),
    )(w4, d4, v3)


_SC_LANES = 16


def _sc_topk_body(corr_hbm, w_hbm, d_hbm, corr_v, w_stage, d_stage, sem):
    wid = lax.axis_index("s") * 2 + lax.axis_index("c")

    pltpu.sync_copy(corr_hbm.at[wid], corr_v)

    def topk_body(i, carry):
        ck, ci = carry
        vals = corr_v[0, pl.ds(i * _SC_LANES, _SC_LANES)]
        idxs = lax.iota(jnp.int32, _SC_LANES) + i * _SC_LANES
        sv, si = plsc.sort_key_val(vals, idxs, descending=True)
        rv = lax.rev(sv, (0,))
        ri = lax.rev(si, (0,))
        take_old = ck >= rv
        mk = jnp.maximum(ck, rv)
        mi = jnp.where(take_old, ci, ri)
        return tuple(plsc.sort_key_val(mk, mi, descending=True))

    ck0 = jnp.full((_SC_LANES,), -jnp.inf, jnp.float32)
    ci0 = jnp.zeros((_SC_LANES,), jnp.int32)
    ck, ci = lax.fori_loop(0, L // _SC_LANES, topk_body, (ck0, ci0))

    m = lax.reduce_max(ck, (0,))
    lanes = lax.iota(jnp.int32, _SC_LANES)
    e = jnp.where(lanes < TOP_K, jnp.exp(ck - m), 0.0)
    s = lax.reduce_sum(e, (0,))
    w_stage[0, :] = e / s
    d_stage[0, :] = ci
    pltpu.sync_copy(w_stage, w_hbm.at[wid])
    pltpu.sync_copy(d_stage, d_hbm.at[wid])


def _sc_topk(corr2):
    mesh = plsc.VectorSubcoreMesh(core_axis_name="c", subcore_axis_name="s")
    f = functools.partial(
        pl.kernel,
        out_type=[jax.ShapeDtypeStruct((BH, 1, _SC_LANES), jnp.float32),
                  jax.ShapeDtypeStruct((BH, 1, _SC_LANES), jnp.int32)],
        mesh=mesh,
        compiler_params=pltpu.CompilerParams(needs_layout_passes=False,
                                             use_tc_tiling_on_sc=False),
        scratch_types=[
            pltpu.VMEM((1, L), jnp.float32),
            pltpu.VMEM((1, _SC_LANES), jnp.float32),
            pltpu.VMEM((1, _SC_LANES), jnp.int32),
            pltpu.SemaphoreType.DMA,
        ],
    )(_sc_topk_body)
    return f(corr2)


def kernel(queries, keys, values, Wq, bq, Wk, bk, Wv, bv, Wo, bo):
    fwc = jnp.asarray(_FWC)
    fws = jnp.asarray(_FWS)
    idc = jnp.asarray(_IDC)
    ids = jnp.asarray(_IDS)
    ed = jnp.asarray(_ED)

    q = _matmul(queries.reshape(B * L, D_MODEL), Wq.T, bq, bk=1024,
                precision=lax.Precision.DEFAULT)
    k = _matmul(keys.reshape(B * L, D_MODEL), Wk.T, bk, bk=1024,
                precision=lax.Precision.DEFAULT)
    v = _matmul(values.reshape(B * L, D_MODEL), Wv.T, bv, bk=1024,
                precision=lax.Precision.DEFAULT)

    q3 = q.reshape(B, L, D_MODEL)
    k3 = k.reshape(B, L, D_MODEL)
    v3 = v.reshape(B, L, D_MODEL)

    qc, qs, kc, ks = _fwd_transforms(fwc, fws, q3, k3)
    sr, si = _cross_spectrum(qc, qs, kc, ks, ed)
    srt = sr.transpose(0, 2, 1)
    sit = si.transpose(0, 2, 1)
    corr = _idft(srt, sit, idc, ids)

    corr2 = corr.reshape(BH, 1, L)
    w2, d2 = _sc_topk(corr2)
    w4 = w2.reshape(B, N_HEADS // 2, 2, _SC_LANES)
    d4 = d2.reshape(B, N_HEADS // 2, 2, _SC_LANES)

    out = _topk_agg(w4, d4, v3)
    out = _matmul(out.reshape(B * L, D_MODEL), Wo.T, bo, bk=1024,
                  precision=lax.Precision.DEFAULT)
    return out.reshape(B, L, D_MODEL)


# speedup vs baseline: 1.7084x; 1.0326x over previous
import functools

import jax
import jax.numpy as jnp
import numpy as np
from jax import lax
from jax.experimental import pallas as pl
from jax.experimental.pallas import tpu as pltpu
from jax.experimental.pallas import tpu_sc as plsc

B = 2
L = 2048
D_MODEL = 1024
N_HEADS = 16
D_K = D_MODEL // N_HEADS
TOP_K = 8
BH = B * N_HEADS
NF = L // 2 + 1
FPAD = 1152

_f = np.arange(FPAD, dtype=np.float64)
_t = np.arange(L, dtype=np.float64)
_theta = (2.0 * np.pi / L) * np.outer(_f, _t)
_mask = (_f < NF)[:, None]
_FWC = np.where(_mask, np.cos(_theta), 0.0).astype(np.float32)
_FWS = np.where(_mask, np.sin(_theta), 0.0).astype(np.float32)
_w = np.where((_f >= 1) & (_f <= NF - 2), 2.0, 1.0) * (_f < NF)
_IDC = (_w[:, None] * np.cos(_theta)).astype(np.float32)
_IDS = (_w[:, None] * np.sin(_theta)).astype(np.float32)
_ED = np.zeros((D_MODEL, N_HEADS), dtype=np.float32)
for _c in range(D_MODEL):
    _ED[_c, _c // D_K] = 1.0 / (L * D_K)

_HI = lax.Precision.HIGHEST


def _mm_kernel(x_ref, y_ref, o_ref, *, precision):
    @pl.when(pl.program_id(2) == 0)
    def _():
        o_ref[...] = jnp.zeros_like(o_ref)

    o_ref[...] += jnp.dot(x_ref[...], y_ref[...],
                          preferred_element_type=jnp.float32,
                          precision=precision)


def _mm_bias_kernel(x_ref, y_ref, b_ref, o_ref, *, precision):
    @pl.when(pl.program_id(2) == 0)
    def _():
        o_ref[...] = jnp.broadcast_to(b_ref[...], o_ref.shape)

    o_ref[...] += jnp.dot(x_ref[...], y_ref[...],
                          preferred_element_type=jnp.float32,
                          precision=precision)


def _matmul(x, y, bias=None, bm=512, bn=512, bk=512, precision=_HI):
    M, K = x.shape
    _, N = y.shape
    bm, bn, bk = min(bm, M), min(bn, N), min(bk, K)
    grid = (M // bm, N // bn, K // bk)
    in_specs = [
        pl.BlockSpec((bm, bk), lambda i, j, k: (i, k)),
        pl.BlockSpec((bk, bn), lambda i, j, k: (k, j)),
    ]
    args = [x, y]
    if bias is None:
        body = functools.partial(_mm_kernel, precision=precision)
    else:
        body = functools.partial(_mm_bias_kernel, precision=precision)
        in_specs.append(pl.BlockSpec((1, bn), lambda i, j, k: (0, j)))
        args.append(bias.reshape(1, N))
    return pl.pallas_call(
        body,
        grid=grid,
        in_specs=in_specs,
        out_specs=pl.BlockSpec((bm, bn), lambda i, j, k: (i, j)),
        out_shape=jax.ShapeDtypeStruct((M, N), jnp.float32),
        compiler_params=pltpu.CompilerParams(
            dimension_semantics=("parallel", "parallel", "arbitrary")),
    )(*args)


def _fwdspec_kernel(fc_ref, fs_ref, q_ref, k_ref, ed_ref, sr_ref, si_ref,
                    qc_acc, qs_acc, kc_acc, ks_acc, *, nk):
    k_id = pl.program_id(3)
    j_id = pl.program_id(2)

    @pl.when(k_id == 0)
    def _():
        qc_acc[...] = jnp.zeros_like(qc_acc)
        qs_acc[...] = jnp.zeros_like(qs_acc)
        kc_acc[...] = jnp.zeros_like(kc_acc)
        ks_acc[...] = jnp.zeros_like(ks_acc)

    fcb, fsb = fc_ref[...], fs_ref[...]
    qb, kb = q_ref[0], k_ref[0]
    dot = functools.partial(jnp.dot, preferred_element_type=jnp.float32,
                            precision=_HI)
    qc_acc[...] += dot(fcb, qb)
    qs_acc[...] += dot(fsb, qb)
    kc_acc[...] += dot(fcb, kb)
    ks_acc[...] += dot(fsb, kb)

    @pl.when(k_id == nk - 1)
    def _():
        @pl.when(j_id == 0)
        def _():
            sr_ref[...] = jnp.zeros_like(sr_ref)
            si_ref[...] = jnp.zeros_like(si_ref)

        qc, qs = qc_acc[...], qs_acc[...]
        kc, ks = kc_acc[...], ks_acc[...]
        ed = ed_ref[...]
        sr_ref[0] += dot(qc * kc + qs * ks, ed)
        si_ref[0] += dot(qc * ks - qs * kc, ed)


def _fwd_spectrum(fwc, fws, q3, k3, ed, bm=384, bn=512, bk=512):
    nk = L // bk
    grid = (B, FPAD // bm, D_MODEL // bn, nk)
    fspec = pl.BlockSpec((bm, bk), lambda b, i, j, k: (i, k))
    xspec = pl.BlockSpec((1, bk, bn), lambda b, i, j, k: (b, k, j))
    return pl.pallas_call(
        functools.partial(_fwdspec_kernel, nk=nk),
        grid=grid,
        in_specs=[fspec, fspec, xspec, xspec,
                  pl.BlockSpec((bn, N_HEADS), lambda b, i, j, k: (j, 0))],
        out_specs=[pl.BlockSpec((1, bm, N_HEADS),
                                lambda b, i, j, k: (b, i, 0))] * 2,
        out_shape=[jax.ShapeDtypeStruct((B, FPAD, N_HEADS), jnp.float32)] * 2,
        scratch_shapes=[pltpu.VMEM((bm, bn), jnp.float32)] * 4,
        compiler_params=pltpu.CompilerParams(
            dimension_semantics=("parallel", "parallel", "arbitrary",
                                 "arbitrary")),
    )(fwc, fws, q3, k3, ed)


def _idft_kernel(srt_ref, sit_ref, idc_ref, ids_ref, o_ref):
    @pl.when(pl.program_id(1) == 0)
    def _():
        o_ref[...] = jnp.zeros_like(o_ref)

    dot = functools.partial(jnp.dot, preferred_element_type=jnp.float32,
                            precision=_HI)
    o_ref[...] += (dot(srt_ref[...], idc_ref[...])
                   - dot(sit_ref[...], ids_ref[...]))


def _idft(srt, sit, idc, ids, bn=512, bk=384):
    grid = (L // bn, FPAD // bk)
    sspec = pl.BlockSpec((BH, bk), lambda j, k: (0, k))
    fspec = pl.BlockSpec((bk, bn), lambda j, k: (k, j))
    return pl.pallas_call(
        _idft_kernel,
        grid=grid,
        in_specs=[sspec, sspec, fspec, fspec],
        out_specs=pl.BlockSpec((BH, bn), lambda j, k: (0, j)),
        out_shape=jax.ShapeDtypeStruct((BH, L), jnp.float32),
        compiler_params=pltpu.CompilerParams(
            dimension_semantics=("parallel", "arbitrary")),
    )(srt, sit, idc, ids)


def _agg_kernel(w_ref, d_ref, v_ref, o_ref, scratch):
    vb = v_ref[0]
    scratch[0:L, :] = vb
    scratch[L:2 * L, :] = vb
    wpair = w_ref[...].reshape(2, _SC_LANES)
    dpair = d_ref[...].reshape(2, _SC_LANES)
    for i in range(2):
        sl = slice(i * D_K, (i + 1) * D_K)
        acc = wpair[i, 0] * scratch[pl.ds(L - dpair[i, 0], L), sl]
        for j in range(1, TOP_K):
            acc += wpair[i, j] * scratch[pl.ds(L - dpair[i, j], L), sl]
        o_ref[0, :, sl] = acc


def _topk_agg(w4, d4, v3):
    return pl.pallas_call(
        _agg_kernel,
        grid=(B, N_HEADS // 2),
        in_specs=[
            pl.BlockSpec((1, 1, 2, _SC_LANES), lambda b, hp: (b, hp, 0, 0)),
            pl.BlockSpec((1, 1, 2, _SC_LANES), lambda b, hp: (b, hp, 0, 0)),
            pl.BlockSpec((1, L, 2 * D_K), lambda b, hp: (b, 0, hp)),
        ],
        out_specs=pl.BlockSpec((1, L, 2 * D_K), lambda b, hp: (b, 0, hp)),
        out_shape=jax.ShapeDtypeStruct((B, L, D_MODEL), jnp.float32),
        scratch_shapes=[pltpu.VMEM((2 * L, 2 * D_K), jnp.float32)],
        compiler_params=pltpu.CompilerParams(
            dimension_semantics=("parallel", "parallel")),
    )(w4, d4, v3)


_SC_LANES = 16


def _sc_topk_body(corr_hbm, w_hbm, d_hbm, corr_v, w_stage, d_stage, sem):
    wid = lax.axis_index("s") * 2 + lax.axis_index("c")

    pltpu.sync_copy(corr_hbm.at[wid], corr_v)

    def topk_body(i, carry):
        ck, ci = carry
        vals = corr_v[0, pl.ds(i * _SC_LANES, _SC_LANES)]
        idxs = lax.iota(jnp.int32, _SC_LANES) + i * _SC_LANES
        sv, si = plsc.sort_key_val(vals, idxs, descending=True)
        rv = lax.rev(sv, (0,))
        ri = lax.rev(si, (0,))
        take_old = ck >= rv
        mk = jnp.maximum(ck, rv)
        mi = jnp.where(take_old, ci, ri)
        return tuple(plsc.sort_key_val(mk, mi, descending=True))

    ck0 = jnp.full((_SC_LANES,), -jnp.inf, jnp.float32)
    ci0 = jnp.zeros((_SC_LANES,), jnp.int32)
    ck, ci = lax.fori_loop(0, L // _SC_LANES, topk_body, (ck0, ci0))

    m = lax.reduce_max(ck, (0,))
    lanes = lax.iota(jnp.int32, _SC_LANES)
    e = jnp.where(lanes < TOP_K, jnp.exp(ck - m), 0.0)
    s = lax.reduce_sum(e, (0,))
    w_stage[0, :] = e / s
    d_stage[0, :] = ci
    pltpu.sync_copy(w_stage, w_hbm.at[wid])
    pltpu.sync_copy(d_stage, d_hbm.at[wid])


def _sc_topk(corr2):
    mesh = plsc.VectorSubcoreMesh(core_axis_name="c", subcore_axis_name="s")
    f = functools.partial(
        pl.kernel,
        out_type=[jax.ShapeDtypeStruct((BH, 1, _SC_LANES), jnp.float32),
                  jax.ShapeDtypeStruct((BH, 1, _SC_LANES), jnp.int32)],
        mesh=mesh,
        compiler_params=pltpu.CompilerParams(needs_layout_passes=False,
                                             use_tc_tiling_on_sc=False),
        scratch_types=[
            pltpu.VMEM((1, L), jnp.float32),
            pltpu.VMEM((1, _SC_LANES), jnp.float32),
            pltpu.VMEM((1, _SC_LANES), jnp.int32),
            pltpu.SemaphoreType.DMA,
        ],
    )(_sc_topk_body)
    return f(corr2)


def kernel(queries, keys, values, Wq, bq, Wk, bk, Wv, bv, Wo, bo):
    fwc = jnp.asarray(_FWC)
    fws = jnp.asarray(_FWS)
    idc = jnp.asarray(_IDC)
    ids = jnp.asarray(_IDS)
    ed = jnp.asarray(_ED)

    q = _matmul(queries.reshape(B * L, D_MODEL), Wq.T, bq, bk=1024,
                precision=lax.Precision.DEFAULT)
    k = _matmul(keys.reshape(B * L, D_MODEL), Wk.T, bk, bk=1024,
                precision=lax.Precision.DEFAULT)
    v = _matmul(values.reshape(B * L, D_MODEL), Wv.T, bv, bk=1024,
                precision=lax.Precision.DEFAULT)

    q3 = q.reshape(B, L, D_MODEL)
    k3 = k.reshape(B, L, D_MODEL)
    v3 = v.reshape(B, L, D_MODEL)

    sr, si = _fwd_spectrum(fwc, fws, q3, k3, ed)
    srt = sr.transpose(0, 2, 1).reshape(BH, FPAD)
    sit = si.transpose(0, 2, 1).reshape(BH, FPAD)
    corr = _idft(srt, sit, idc, ids)

    corr2 = corr.reshape(BH, 1, L)
    w2, d2 = _sc_topk(corr2)
    w4 = w2.reshape(B, N_HEADS // 2, 2, _SC_LANES)
    d4 = d2.reshape(B, N_HEADS // 2, 2, _SC_LANES)

    out = _topk_agg(w4, d4, v3)
    out = _matmul(out.reshape(B * L, D_MODEL), Wo.T, bo, bk=1024,
                  precision=lax.Precision.DEFAULT)
    return out.reshape(B, L, D_MODEL)


# speedup vs baseline: 1.8786x; 1.0997x over previous
import functools

import jax
import jax.numpy as jnp
import numpy as np
from jax import lax
from jax.experimental import pallas as pl
from jax.experimental.pallas import tpu as pltpu
from jax.experimental.pallas import tpu_sc as plsc

B = 2
L = 2048
D_MODEL = 1024
N_HEADS = 16
D_K = D_MODEL // N_HEADS
TOP_K = 8
BH = B * N_HEADS
NF = L // 2 + 1
FPAD = 1152

_f = np.arange(FPAD, dtype=np.float64)
_t = np.arange(L, dtype=np.float64)
_theta = (2.0 * np.pi / L) * np.outer(_f, _t)
_mask = (_f < NF)[:, None]
_FWC = np.where(_mask, np.cos(_theta), 0.0).astype(np.float32)
_FWS = np.where(_mask, np.sin(_theta), 0.0).astype(np.float32)
_w = np.where((_f >= 1) & (_f <= NF - 2), 2.0, 1.0) * (_f < NF)
_IDC = (_w[:, None] * np.cos(_theta)).astype(np.float32)
_IDS = (_w[:, None] * np.sin(_theta)).astype(np.float32)
_ED = np.zeros((D_MODEL, N_HEADS), dtype=np.float32)
for _c in range(D_MODEL):
    _ED[_c, _c // D_K] = 1.0 / (L * D_K)

_HI = lax.Precision.HIGHEST


def _mm_kernel(x_ref, y_ref, o_ref, *, precision):
    @pl.when(pl.program_id(2) == 0)
    def _():
        o_ref[...] = jnp.zeros_like(o_ref)

    o_ref[...] += jnp.dot(x_ref[...], y_ref[...],
                          preferred_element_type=jnp.float32,
                          precision=precision)


def _mm_bias_kernel(x_ref, y_ref, b_ref, o_ref, *, precision):
    @pl.when(pl.program_id(2) == 0)
    def _():
        o_ref[...] = jnp.broadcast_to(b_ref[...], o_ref.shape)

    o_ref[...] += jnp.dot(x_ref[...], y_ref[...],
                          preferred_element_type=jnp.float32,
                          precision=precision)


def _matmul(x, y, bias=None, bm=512, bn=512, bk=512, precision=_HI):
    M, K = x.shape
    _, N = y.shape
    bm, bn, bk = min(bm, M), min(bn, N), min(bk, K)
    grid = (M // bm, N // bn, K // bk)
    in_specs = [
        pl.BlockSpec((bm, bk), lambda i, j, k: (i, k)),
        pl.BlockSpec((bk, bn), lambda i, j, k: (k, j)),
    ]
    args = [x, y]
    if bias is None:
        body = functools.partial(_mm_kernel, precision=precision)
    else:
        body = functools.partial(_mm_bias_kernel, precision=precision)
        in_specs.append(pl.BlockSpec((1, bn), lambda i, j, k: (0, j)))
        args.append(bias.reshape(1, N))
    return pl.pallas_call(
        body,
        grid=grid,
        in_specs=in_specs,
        out_specs=pl.BlockSpec((bm, bn), lambda i, j, k: (i, j)),
        out_shape=jax.ShapeDtypeStruct((M, N), jnp.float32),
        compiler_params=pltpu.CompilerParams(
            dimension_semantics=("parallel", "parallel", "arbitrary")),
    )(*args)


def _fwdspec_kernel(fc_ref, fs_ref, q_ref, k_ref, ed_ref, sr_ref, si_ref,
                    qc_acc, qs_acc, kc_acc, ks_acc, *, nk):
    k_id = pl.program_id(3)
    j_id = pl.program_id(2)

    @pl.when(k_id == 0)
    def _():
        qc_acc[...] = jnp.zeros_like(qc_acc)
        qs_acc[...] = jnp.zeros_like(qs_acc)
        kc_acc[...] = jnp.zeros_like(kc_acc)
        ks_acc[...] = jnp.zeros_like(ks_acc)

    fcb, fsb = fc_ref[...], fs_ref[...]
    qb, kb = q_ref[0], k_ref[0]
    dot = functools.partial(jnp.dot, preferred_element_type=jnp.float32,
                            precision=_HI)
    qc_acc[...] += dot(fcb, qb)
    qs_acc[...] += dot(fsb, qb)
    kc_acc[...] += dot(fcb, kb)
    ks_acc[...] += dot(fsb, kb)

    @pl.when(k_id == nk - 1)
    def _():
        @pl.when(j_id == 0)
        def _():
            sr_ref[...] = jnp.zeros_like(sr_ref)
            si_ref[...] = jnp.zeros_like(si_ref)

        qc, qs = qc_acc[...], qs_acc[...]
        kc, ks = kc_acc[...], ks_acc[...]
        ed = ed_ref[...]
        sr_ref[0] += dot(qc * kc + qs * ks, ed)
        si_ref[0] += dot(qc * ks - qs * kc, ed)


def _fwd_spectrum(fwc, fws, q3, k3, ed, bm=384, bn=512, bk=512):
    nk = L // bk
    grid = (B, FPAD // bm, D_MODEL // bn, nk)
    fspec = pl.BlockSpec((bm, bk), lambda b, i, j, k: (i, k))
    xspec = pl.BlockSpec((1, bk, bn), lambda b, i, j, k: (b, k, j))
    return pl.pallas_call(
        functools.partial(_fwdspec_kernel, nk=nk),
        grid=grid,
        in_specs=[fspec, fspec, xspec, xspec,
                  pl.BlockSpec((bn, N_HEADS), lambda b, i, j, k: (j, 0))],
        out_specs=[pl.BlockSpec((1, bm, N_HEADS),
                                lambda b, i, j, k: (b, i, 0))] * 2,
        out_shape=[jax.ShapeDtypeStruct((B, FPAD, N_HEADS), jnp.float32)] * 2,
        scratch_shapes=[pltpu.VMEM((bm, bn), jnp.float32)] * 4,
        compiler_params=pltpu.CompilerParams(
            dimension_semantics=("parallel", "parallel", "arbitrary",
                                 "arbitrary")),
    )(fwc, fws, q3, k3, ed)


def _idft_kernel(srt_ref, sit_ref, idc_ref, ids_ref, o_ref):
    @pl.when(pl.program_id(1) == 0)
    def _():
        o_ref[...] = jnp.zeros_like(o_ref)

    dot = functools.partial(jnp.dot, preferred_element_type=jnp.float32,
                            precision=_HI)
    o_ref[...] += (dot(srt_ref[...], idc_ref[...])
                   - dot(sit_ref[...], ids_ref[...]))


def _idft(srt, sit, idc, ids, bn=512, bk=384):
    grid = (L // bn, FPAD // bk)
    sspec = pl.BlockSpec((BH, bk), lambda j, k: (0, k))
    fspec = pl.BlockSpec((bk, bn), lambda j, k: (k, j))
    return pl.pallas_call(
        _idft_kernel,
        grid=grid,
        in_specs=[sspec, sspec, fspec, fspec],
        out_specs=pl.BlockSpec((BH, bn), lambda j, k: (0, j)),
        out_shape=jax.ShapeDtypeStruct((BH, L), jnp.float32),
        compiler_params=pltpu.CompilerParams(
            dimension_semantics=("parallel", "arbitrary")),
    )(srt, sit, idc, ids)


def _agg_kernel(w_ref, d_ref, v_ref, o_ref, scratch):
    vb = v_ref[0]
    scratch[0:L, :] = vb
    scratch[L:2 * L, :] = vb
    wpair = w_ref[...].reshape(2, _SC_LANES)
    dpair = d_ref[...].reshape(2, _SC_LANES)
    for i in range(2):
        sl = slice(i * D_K, (i + 1) * D_K)
        acc = wpair[i, 0] * scratch[pl.ds(L - dpair[i, 0], L), sl]
        for j in range(1, TOP_K):
            acc += wpair[i, j] * scratch[pl.ds(L - dpair[i, j], L), sl]
        o_ref[0, :, sl] = acc


def _topk_agg(w4, d4, v3):
    return pl.pallas_call(
        _agg_kernel,
        grid=(B, N_HEADS // 2),
        in_specs=[
            pl.BlockSpec((1, 1, 2, _SC_LANES), lambda b, hp: (b, hp, 0, 0)),
            pl.BlockSpec((1, 1, 2, _SC_LANES), lambda b, hp: (b, hp, 0, 0)),
            pl.BlockSpec((1, L, 2 * D_K), lambda b, hp: (b, 0, hp)),
        ],
        out_specs=pl.BlockSpec((1, L, 2 * D_K), lambda b, hp: (b, 0, hp)),
        out_shape=jax.ShapeDtypeStruct((B, L, D_MODEL), jnp.float32),
        scratch_shapes=[pltpu.VMEM((2 * L, 2 * D_K), jnp.float32)],
        compiler_params=pltpu.CompilerParams(
            dimension_semantics=("parallel", "parallel")),
    )(w4, d4, v3)


_SC_LANES = 16


def _sc_topk_body(corr_hbm, w_hbm, d_hbm, corr_v, w_stage, d_stage, sem):
    wid = lax.axis_index("s") * 2 + lax.axis_index("c")

    pltpu.sync_copy(corr_hbm.at[wid], corr_v)

    def topk_body(i, carry):
        ck, ci = carry
        vals = corr_v[0, pl.ds(i * _SC_LANES, _SC_LANES)]
        idxs = lax.iota(jnp.int32, _SC_LANES) + i * _SC_LANES
        sv, si = plsc.sort_key_val(vals, idxs, descending=True)
        rv = lax.rev(sv, (0,))
        ri = lax.rev(si, (0,))
        take_old = ck >= rv
        mk = jnp.maximum(ck, rv)
        mi = jnp.where(take_old, ci, ri)
        return tuple(plsc.sort_key_val(mk, mi, descending=True))

    ck0 = jnp.full((_SC_LANES,), -jnp.inf, jnp.float32)
    ci0 = jnp.zeros((_SC_LANES,), jnp.int32)
    ck, ci = lax.fori_loop(0, L // _SC_LANES, topk_body, (ck0, ci0))

    m = lax.reduce_max(ck, (0,))
    lanes = lax.iota(jnp.int32, _SC_LANES)
    e = jnp.where(lanes < TOP_K, jnp.exp(ck - m), 0.0)
    s = lax.reduce_sum(e, (0,))
    w_stage[0, :] = e / s
    d_stage[0, :] = ci
    pltpu.sync_copy(w_stage, w_hbm.at[wid])
    pltpu.sync_copy(d_stage, d_hbm.at[wid])


def _sc_topk(corr2):
    mesh = plsc.VectorSubcoreMesh(core_axis_name="c", subcore_axis_name="s")
    f = functools.partial(
        pl.kernel,
        out_type=[jax.ShapeDtypeStruct((BH, 1, _SC_LANES), jnp.float32),
                  jax.ShapeDtypeStruct((BH, 1, _SC_LANES), jnp.int32)],
        mesh=mesh,
        compiler_params=pltpu.CompilerParams(needs_layout_passes=False,
                                             use_tc_tiling_on_sc=False),
        scratch_types=[
            pltpu.VMEM((1, L), jnp.float32),
            pltpu.VMEM((1, _SC_LANES), jnp.float32),
            pltpu.VMEM((1, _SC_LANES), jnp.int32),
            pltpu.SemaphoreType.DMA,
        ],
    )(_sc_topk_body)
    return f(corr2)


def kernel(queries, keys, values, Wq, bq, Wk, bk, Wv, bv, Wo, bo):
    fwc = jnp.asarray(_FWC)
    fws = jnp.asarray(_FWS)
    idc = jnp.asarray(_IDC)
    ids = jnp.asarray(_IDS)
    ed = jnp.asarray(_ED)

    q = _matmul(queries.reshape(B * L, D_MODEL), Wq.T, bq, bn=1024, bk=1024,
                precision=lax.Precision.DEFAULT)
    k = _matmul(keys.reshape(B * L, D_MODEL), Wk.T, bk, bn=1024, bk=1024,
                precision=lax.Precision.DEFAULT)
    v = _matmul(values.reshape(B * L, D_MODEL), Wv.T, bv, bn=1024, bk=1024,
                precision=lax.Precision.DEFAULT)

    q3 = q.reshape(B, L, D_MODEL)
    k3 = k.reshape(B, L, D_MODEL)
    v3 = v.reshape(B, L, D_MODEL)

    sr, si = _fwd_spectrum(fwc, fws, q3, k3, ed)
    srt = sr.transpose(0, 2, 1).reshape(BH, FPAD)
    sit = si.transpose(0, 2, 1).reshape(BH, FPAD)
    corr = _idft(srt, sit, idc, ids)

    corr2 = corr.reshape(BH, 1, L)
    w2, d2 = _sc_topk(corr2)
    w4 = w2.reshape(B, N_HEADS // 2, 2, _SC_LANES)
    d4 = d2.reshape(B, N_HEADS // 2, 2, _SC_LANES)

    out = _topk_agg(w4, d4, v3)
    out = _matmul(out.reshape(B * L, D_MODEL), Wo.T, bo, bn=1024, bk=1024,
                  precision=lax.Precision.DEFAULT)
    return out.reshape(B, L, D_MODEL)


# speedup vs baseline: 1.9197x; 1.0219x over previous
import functools

import jax
import jax.numpy as jnp
import numpy as np
from jax import lax
from jax.experimental import pallas as pl
from jax.experimental.pallas import tpu as pltpu
from jax.experimental.pallas import tpu_sc as plsc

B = 2
L = 2048
D_MODEL = 1024
N_HEADS = 16
D_K = D_MODEL // N_HEADS
TOP_K = 8
BH = B * N_HEADS
NF = L // 2 + 1
FPAD = 1152

_f = np.arange(FPAD, dtype=np.float64)
_t = np.arange(L, dtype=np.float64)
_theta = (2.0 * np.pi / L) * np.outer(_f, _t)
_mask = (_f < NF)[:, None]
_FWC = np.where(_mask, np.cos(_theta), 0.0).astype(np.float32)
_FWS = np.where(_mask, np.sin(_theta), 0.0).astype(np.float32)
_w = np.where((_f >= 1) & (_f <= NF - 2), 2.0, 1.0) * (_f < NF)
_IDC = (_w[:, None] * np.cos(_theta)).astype(np.float32)
_IDS = (_w[:, None] * np.sin(_theta)).astype(np.float32)
_ED = np.zeros((D_MODEL, N_HEADS), dtype=np.float32)
for _c in range(D_MODEL):
    _ED[_c, _c // D_K] = 1.0 / (L * D_K)

_HI = lax.Precision.HIGHEST


def _mm_kernel(x_ref, y_ref, o_ref, *, precision):
    @pl.when(pl.program_id(2) == 0)
    def _():
        o_ref[...] = jnp.zeros_like(o_ref)

    o_ref[...] += jnp.dot(x_ref[...], y_ref[...],
                          preferred_element_type=jnp.float32,
                          precision=precision)


def _mm_bias_kernel(x_ref, y_ref, b_ref, o_ref, *, precision):
    @pl.when(pl.program_id(2) == 0)
    def _():
        o_ref[...] = jnp.broadcast_to(b_ref[...], o_ref.shape)

    o_ref[...] += jnp.dot(x_ref[...], y_ref[...],
                          preferred_element_type=jnp.float32,
                          precision=precision)


def _matmul(x, y, bias=None, bm=512, bn=512, bk=512, precision=_HI):
    M, K = x.shape
    _, N = y.shape
    bm, bn, bk = min(bm, M), min(bn, N), min(bk, K)
    grid = (M // bm, N // bn, K // bk)
    in_specs = [
        pl.BlockSpec((bm, bk), lambda i, j, k: (i, k)),
        pl.BlockSpec((bk, bn), lambda i, j, k: (k, j)),
    ]
    args = [x, y]
    if bias is None:
        body = functools.partial(_mm_kernel, precision=precision)
    else:
        body = functools.partial(_mm_bias_kernel, precision=precision)
        in_specs.append(pl.BlockSpec((1, bn), lambda i, j, k: (0, j)))
        args.append(bias.reshape(1, N))
    return pl.pallas_call(
        body,
        grid=grid,
        in_specs=in_specs,
        out_specs=pl.BlockSpec((bm, bn), lambda i, j, k: (i, j)),
        out_shape=jax.ShapeDtypeStruct((M, N), jnp.float32),
        compiler_params=pltpu.CompilerParams(
            dimension_semantics=("parallel", "parallel", "arbitrary")),
    )(*args)


def _fwdspec_kernel(fc_ref, fs_ref, q_ref, k_ref, ed_ref, sr_ref, si_ref,
                    qc_acc, qs_acc, kc_acc, ks_acc, *, nk):
    k_id = pl.program_id(3)
    j_id = pl.program_id(2)

    @pl.when(k_id == 0)
    def _():
        qc_acc[...] = jnp.zeros_like(qc_acc)
        qs_acc[...] = jnp.zeros_like(qs_acc)
        kc_acc[...] = jnp.zeros_like(kc_acc)
        ks_acc[...] = jnp.zeros_like(ks_acc)

    fcb, fsb = fc_ref[...], fs_ref[...]
    qb, kb = q_ref[0], k_ref[0]
    dot = functools.partial(jnp.dot, preferred_element_type=jnp.float32,
                            precision=_HI)
    qc_acc[...] += dot(fcb, qb)
    qs_acc[...] += dot(fsb, qb)
    kc_acc[...] += dot(fcb, kb)
    ks_acc[...] += dot(fsb, kb)

    @pl.when(k_id == nk - 1)
    def _():
        @pl.when(j_id == 0)
        def _():
            sr_ref[...] = jnp.zeros_like(sr_ref)
            si_ref[...] = jnp.zeros_like(si_ref)

        qc, qs = qc_acc[...], qs_acc[...]
        kc, ks = kc_acc[...], ks_acc[...]
        ed = ed_ref[...]
        sr_ref[0] += dot(qc * kc + qs * ks, ed)
        si_ref[0] += dot(qc * ks - qs * kc, ed)


def _fwd_spectrum(fwc, fws, q3, k3, ed, bm=384, bn=1024, bk=512):
    nk = L // bk
    grid = (B, FPAD // bm, D_MODEL // bn, nk)
    fspec = pl.BlockSpec((bm, bk), lambda b, i, j, k: (i, k))
    xspec = pl.BlockSpec((1, bk, bn), lambda b, i, j, k: (b, k, j))
    return pl.pallas_call(
        functools.partial(_fwdspec_kernel, nk=nk),
        grid=grid,
        in_specs=[fspec, fspec, xspec, xspec,
                  pl.BlockSpec((bn, N_HEADS), lambda b, i, j, k: (j, 0))],
        out_specs=[pl.BlockSpec((1, bm, N_HEADS),
                                lambda b, i, j, k: (b, i, 0))] * 2,
        out_shape=[jax.ShapeDtypeStruct((B, FPAD, N_HEADS), jnp.float32)] * 2,
        scratch_shapes=[pltpu.VMEM((bm, bn), jnp.float32)] * 4,
        compiler_params=pltpu.CompilerParams(
            dimension_semantics=("parallel", "parallel", "arbitrary",
                                 "arbitrary")),
    )(fwc, fws, q3, k3, ed)


def _idft_kernel(srt_ref, sit_ref, idc_ref, ids_ref, o_ref):
    @pl.when(pl.program_id(1) == 0)
    def _():
        o_ref[...] = jnp.zeros_like(o_ref)

    dot = functools.partial(jnp.dot, preferred_element_type=jnp.float32,
                            precision=_HI)
    o_ref[...] += (dot(srt_ref[...], idc_ref[...])
                   - dot(sit_ref[...], ids_ref[...]))


def _idft(srt, sit, idc, ids, bn=512, bk=384):
    grid = (L // bn, FPAD // bk)
    sspec = pl.BlockSpec((BH, bk), lambda j, k: (0, k))
    fspec = pl.BlockSpec((bk, bn), lambda j, k: (k, j))
    return pl.pallas_call(
        _idft_kernel,
        grid=grid,
        in_specs=[sspec, sspec, fspec, fspec],
        out_specs=pl.BlockSpec((BH, bn), lambda j, k: (0, j)),
        out_shape=jax.ShapeDtypeStruct((BH, L), jnp.float32),
        compiler_params=pltpu.CompilerParams(
            dimension_semantics=("parallel", "arbitrary")),
    )(srt, sit, idc, ids)


def _agg_kernel(w_ref, d_ref, v_ref, o_ref, scratch):
    vb = v_ref[0]
    scratch[0:L, :] = vb
    scratch[L:2 * L, :] = vb
    wpair = w_ref[...].reshape(2, _SC_LANES)
    dpair = d_ref[...].reshape(2, _SC_LANES)
    for i in range(2):
        sl = slice(i * D_K, (i + 1) * D_K)
        acc = wpair[i, 0] * scratch[pl.ds(L - dpair[i, 0], L), sl]
        for j in range(1, TOP_K):
            acc += wpair[i, j] * scratch[pl.ds(L - dpair[i, j], L), sl]
        o_ref[0, :, sl] = acc


def _topk_agg(w4, d4, v3):
    return pl.pallas_call(
        _agg_kernel,
        grid=(B, N_HEADS // 2),
        in_specs=[
            pl.BlockSpec((1, 1, 2, _SC_LANES), lambda b, hp: (b, hp, 0, 0)),
            pl.BlockSpec((1, 1, 2, _SC_LANES), lambda b, hp: (b, hp, 0, 0)),
            pl.BlockSpec((1, L, 2 * D_K), lambda b, hp: (b, 0, hp)),
        ],
        out_specs=pl.BlockSpec((1, L, 2 * D_K), lambda b, hp: (b, 0, hp)),
        out_shape=jax.ShapeDtypeStruct((B, L, D_MODEL), jnp.float32),
        scratch_shapes=[pltpu.VMEM((2 * L, 2 * D_K), jnp.float32)],
        compiler_params=pltpu.CompilerParams(
            dimension_semantics=("parallel", "parallel")),
    )(w4, d4, v3)


_SC_LANES = 16


def _sc_topk_body(corr_hbm, w_hbm, d_hbm, corr_v, w_stage, d_stage, sem):
    wid = lax.axis_index("s") * 2 + lax.axis_index("c")

    pltpu.sync_copy(corr_hbm.at[wid], corr_v)

    def topk_body(i, carry):
        ck, ci = carry
        vals = corr_v[0, pl.ds(i * _SC_LANES, _SC_LANES)]
        idxs = lax.iota(jnp.int32, _SC_LANES) + i * _SC_LANES
        sv, si = plsc.sort_key_val(vals, idxs, descending=True)
        rv = lax.rev(sv, (0,))
        ri = lax.rev(si, (0,))
        take_old = ck >= rv
        mk = jnp.maximum(ck, rv)
        mi = jnp.where(take_old, ci, ri)
        return tuple(plsc.sort_key_val(mk, mi, descending=True))

    ck0 = jnp.full((_SC_LANES,), -jnp.inf, jnp.float32)
    ci0 = jnp.zeros((_SC_LANES,), jnp.int32)
    ck, ci = lax.fori_loop(0, L // _SC_LANES, topk_body, (ck0, ci0))

    m = lax.reduce_max(ck, (0,))
    lanes = lax.iota(jnp.int32, _SC_LANES)
    e = jnp.where(lanes < TOP_K, jnp.exp(ck - m), 0.0)
    s = lax.reduce_sum(e, (0,))
    w_stage[0, :] = e / s
    d_stage[0, :] = ci
    pltpu.sync_copy(w_stage, w_hbm.at[wid])
    pltpu.sync_copy(d_stage, d_hbm.at[wid])


def _sc_topk(corr2):
    mesh = plsc.VectorSubcoreMesh(core_axis_name="c", subcore_axis_name="s")
    f = functools.partial(
        pl.kernel,
        out_type=[jax.ShapeDtypeStruct((BH, 1, _SC_LANES), jnp.float32),
                  jax.ShapeDtypeStruct((BH, 1, _SC_LANES), jnp.int32)],
        mesh=mesh,
        compiler_params=pltpu.CompilerParams(needs_layout_passes=False,
                                             use_tc_tiling_on_sc=False),
        scratch_types=[
            pltpu.VMEM((1, L), jnp.float32),
            pltpu.VMEM((1, _SC_LANES), jnp.float32),
            pltpu.VMEM((1, _SC_LANES), jnp.int32),
            pltpu.SemaphoreType.DMA,
        ],
    )(_sc_topk_body)
    return f(corr2)


def kernel(queries, keys, values, Wq, bq, Wk, bk, Wv, bv, Wo, bo):
    fwc = jnp.asarray(_FWC)
    fws = jnp.asarray(_FWS)
    idc = jnp.asarray(_IDC)
    ids = jnp.asarray(_IDS)
    ed = jnp.asarray(_ED)

    q = _matmul(queries.reshape(B * L, D_MODEL), Wq.T, bq, bn=1024, bk=1024,
                precision=lax.Precision.DEFAULT)
    k = _matmul(keys.reshape(B * L, D_MODEL), Wk.T, bk, bn=1024, bk=1024,
                precision=lax.Precision.DEFAULT)
    v = _matmul(values.reshape(B * L, D_MODEL), Wv.T, bv, bn=1024, bk=1024,
                precision=lax.Precision.DEFAULT)

    q3 = q.reshape(B, L, D_MODEL)
    k3 = k.reshape(B, L, D_MODEL)
    v3 = v.reshape(B, L, D_MODEL)

    sr, si = _fwd_spectrum(fwc, fws, q3, k3, ed)
    srt = sr.transpose(0, 2, 1).reshape(BH, FPAD)
    sit = si.transpose(0, 2, 1).reshape(BH, FPAD)
    corr = _idft(srt, sit, idc, ids)

    corr2 = corr.reshape(BH, 1, L)
    w2, d2 = _sc_topk(corr2)
    w4 = w2.reshape(B, N_HEADS // 2, 2, _SC_LANES)
    d4 = d2.reshape(B, N_HEADS // 2, 2, _SC_LANES)

    out = _topk_agg(w4, d4, v3)
    out = _matmul(out.reshape(B * L, D_MODEL), Wo.T, bo, bn=1024, bk=1024,
                  precision=lax.Precision.DEFAULT)
    return out.reshape(B, L, D_MODEL)
